# baseline (speedup 1.0000x reference)
import functools

import jax
import jax.numpy as jnp
from jax import lax
from jax.experimental import pallas as pl
from jax.experimental.pallas import tpu as pltpu
from jax.experimental.pallas import tpu_sc as plsc

N = 100000
E = 1600000
M = 4096
D_IN = 32
H = 64
D_OUT = 128
EPS = 1e-5

NC = 2
NS = 16
NW = NC * NS

N_PAD = 100352
TILE_N = N_PAD // NS
M_PAD = 4224
TILE_M = M_PAD // NS

EDGE_K = 800
E_PER_TILE = E // NS
N_CHUNKS = E_PER_TILE // EDGE_K
DEG_K = 2000
E_PER_W = E // NW
POOL_ROWS = N_PAD // NW
POOL_K = 1568

_MESH = plsc.VectorSubcoreMesh(core_axis_name="c", subcore_axis_name="s")
_SC_PARAMS = pltpu.CompilerParams(use_tc_tiling_on_sc=False)


def _f32(*shape):
    return jax.ShapeDtypeStruct(shape, jnp.float32)


@functools.partial(
    pl.kernel,
    out_type=_f32(NC, N_PAD),
    mesh=_MESH,
    compiler_params=_SC_PARAMS,
    scratch_types=[
        pltpu.VMEM((DEG_K,), jnp.int32),
        pltpu.VMEM((DEG_K,), jnp.float32),
        pltpu.VMEM_SHARED((N_PAD,), jnp.float32),
    ],
)
def _deg_kernel(ei_hbm, zeros_hbm, ones_hbm, out_hbm, idx_v, ones_v, acc):
    c = lax.axis_index("c")
    s = lax.axis_index("s")
    w = c * NS + s
    pltpu.sync_copy(zeros_hbm.at[pl.ds(s * TILE_N, TILE_N)],
                    acc.at[pl.ds(s * TILE_N, TILE_N)])
    pltpu.sync_copy(ones_hbm.at[pl.ds(0, DEG_K)], ones_v)
    plsc.subcore_barrier()

    @pl.loop(0, E_PER_W // DEG_K)
    def _(i):
        base = w * E_PER_W + i * DEG_K
        pltpu.sync_copy(ei_hbm.at[1, pl.ds(base, DEG_K)], idx_v)
        pltpu.sync_copy(ones_v, acc.at[idx_v], add=True)

    plsc.subcore_barrier()
    pltpu.sync_copy(acc.at[pl.ds(s * TILE_N, TILE_N)],
                    out_hbm.at[c, pl.ds(s * TILE_N, TILE_N)])


def _make_edge_agg(n_passes):
    n_chunks = n_passes * NC

    @functools.partial(
        pl.kernel,
        out_type=_f32(n_chunks, N_PAD, 16),
        mesh=_MESH,
        compiler_params=_SC_PARAMS,
        scratch_types=[
            pltpu.VMEM((2, EDGE_K), jnp.int32),
            pltpu.VMEM((2, EDGE_K), jnp.int32),
            pltpu.VMEM((2, EDGE_K, 16), jnp.float32),
            pltpu.VMEM_SHARED((N_PAD, 16), jnp.float32),
            pltpu.SemaphoreType.DMA((2,)),
            pltpu.SemaphoreType.DMA((2,)),
            pltpu.SemaphoreType.DMA((2,)),
            pltpu.SemaphoreType.DMA((2,)),
        ],
    )
    def agg_kernel(idx_hbm, ei_hbm, zeros_hbm, table_hbm, out_hbm,
                   sidx, didx, rows, acc, sem_si, sem_di, sem_g, sem_s):
        c = lax.axis_index("c")
        s = lax.axis_index("s")

        @pl.loop(0, n_passes)
        def _(p):
            k = c * n_passes + p
            table = table_hbm

            def base_of(i):
                return s * E_PER_TILE + jnp.minimum(i, N_CHUNKS - 1) * EDGE_K

            def idx_load(i, b):
                pltpu.async_copy(idx_hbm.at[k, pl.ds(base_of(i), EDGE_K)],
                                 sidx.at[b], sem_si.at[b])
                pltpu.async_copy(ei_hbm.at[1, pl.ds(base_of(i), EDGE_K)],
                                 didx.at[b], sem_di.at[b])
                pltpu.make_async_copy(idx_hbm.at[k, pl.ds(base_of(i), EDGE_K)],
                                      sidx.at[b], sem_si.at[b]).wait()
                pltpu.make_async_copy(
                    ei_hbm.at[1, pl.ds(base_of(i), EDGE_K)],
                    didx.at[b], sem_di.at[b]).wait()

            def g_start(b):
                pltpu.async_copy(table.at[sidx.at[b]], rows.at[b],
                                 sem_g.at[b])

            def g_wait(b):
                pltpu.make_async_copy(table.at[sidx.at[b]], rows.at[b],
                                      sem_g.at[b]).wait()

            def s_start(b):
                pltpu.async_copy(rows.at[b], acc.at[didx.at[b]],
                                 sem_s.at[b], add=True)

            def s_wait(b):
                pltpu.make_async_copy(rows.at[b], acc.at[didx.at[b]],
                                      sem_s.at[b]).wait()

            pltpu.sync_copy(zeros_hbm.at[pl.ds(s * TILE_N, TILE_N), :],
                            acc.at[pl.ds(s * TILE_N, TILE_N), :])
            plsc.subcore_barrier()
            pltpu.sync_copy(zeros_hbm.at[pl.ds(0, EDGE_K), :], rows.at[1])
            pltpu.sync_copy(ei_hbm.at[1, pl.ds(s * E_PER_TILE, EDGE_K)],
                            didx.at[1])
            s_start(1)
            idx_load(0, 0)
            g_start(0)

            @pl.loop(0, N_CHUNKS // 2)
            def _(j):
                i0 = 2 * j
                g_wait(0)
                s_start(0)
                s_wait(1)
                idx_load(i0 + 1, 1)
                g_start(1)
                g_wait(1)
                s_start(1)
                s_wait(0)
                idx_load(i0 + 2, 0)
                g_start(0)

            g_wait(0)
            s_start(0)
            s_wait(1)
            s_wait(0)
            plsc.subcore_barrier()
            pltpu.sync_copy(acc.at[pl.ds(s * TILE_N, TILE_N), :],
                            out_hbm.at[k, pl.ds(s * TILE_N, TILE_N), :])

    return agg_kernel


_agg2 = _make_edge_agg(1)
_agg4 = _make_edge_agg(2)


@functools.partial(
    pl.kernel,
    out_type=(_f32(NC, M_PAD, H), _f32(NC, M_PAD)),
    mesh=_MESH,
    compiler_params=_SC_PARAMS,
    scratch_types=[
        pltpu.VMEM((POOL_K,), jnp.int32),
        pltpu.VMEM((POOL_K, H), jnp.float32),
        pltpu.VMEM((DEG_K,), jnp.float32),
        pltpu.VMEM_SHARED((M_PAD, H), jnp.float32),
        pltpu.VMEM_SHARED((M_PAD,), jnp.float32),
    ],
)
def _pool_kernel(h2_hbm, bid_hbm, zeros2_hbm, zeros1_hbm, ones_hbm,
                 outs_hbm, outc_hbm, idx_v, rows_v, ones_v, accs, accc):
    c = lax.axis_index("c")
    s = lax.axis_index("s")
    w = c * NS + s
    pltpu.sync_copy(zeros2_hbm.at[pl.ds(s * TILE_M, TILE_M), :],
                    accs.at[pl.ds(s * TILE_M, TILE_M), :])
    pltpu.sync_copy(zeros1_hbm.at[pl.ds(s * TILE_M, TILE_M)],
                    accc.at[pl.ds(s * TILE_M, TILE_M)])
    pltpu.sync_copy(ones_hbm.at[pl.ds(0, DEG_K)], ones_v)
    plsc.subcore_barrier()

    @pl.loop(0, POOL_ROWS // POOL_K)
    def _(j):
        base = w * POOL_ROWS + j * POOL_K
        pltpu.sync_copy(h2_hbm.at[pl.ds(base, POOL_K), :], rows_v)
        pltpu.sync_copy(bid_hbm.at[pl.ds(base, POOL_K)], idx_v)
        pltpu.sync_copy(rows_v, accs.at[idx_v], add=True)
        pltpu.sync_copy(ones_v.at[pl.ds(0, POOL_K)], accc.at[idx_v], add=True)

    plsc.subcore_barrier()
    pltpu.sync_copy(accs.at[pl.ds(s * TILE_M, TILE_M), :],
                    outs_hbm.at[c, pl.ds(s * TILE_M, TILE_M), :])
    pltpu.sync_copy(accc.at[pl.ds(s * TILE_M, TILE_M)],
                    outc_hbm.at[c, pl.ds(s * TILE_M, TILE_M)])


BLK = 2048
GRID_N = N_PAD // BLK


def _ln(h, g, b):
    mu = jnp.mean(h, axis=-1, keepdims=True)
    var = jnp.mean((h - mu) ** 2, axis=-1, keepdims=True)
    return (h - mu) * lax.rsqrt(var + EPS) * g + b


def _prep_body(counts_ref, x_ref, x2_ref, dis_ref):
    cnt = counts_ref[0, :] + counts_ref[1, :]
    dis = lax.rsqrt(cnt + 1.0)
    d2 = jnp.reshape(dis, (BLK, 1))
    x2_ref[...] = x_ref[...] * d2
    dis_ref[...] = d2


def _prep(counts, x):
    return pl.pallas_call(
        _prep_body,
        grid=(GRID_N,),
        in_specs=[
            pl.BlockSpec((NC, BLK), lambda i: (0, i)),
            pl.BlockSpec((BLK, D_IN), lambda i: (i, 0)),
        ],
        out_specs=[
            pl.BlockSpec((BLK, D_IN), lambda i: (i, 0)),
            pl.BlockSpec((BLK, 1), lambda i: (i, 0)),
        ],
        out_shape=[_f32(N_PAD, D_IN), _f32(N_PAD, 1)],
    )(counts, x)


E_ROWS = 3125
E_COLS = 512
E_CBLK = 128


def _idx_prep_body(s_ref, i2_ref, i4_ref):
    s = s_ref[...]
    s2 = s * 2
    s4 = s * 4
    i2_ref[...] = jnp.stack([s2, s2 + 1], axis=0)
    i4_ref[...] = jnp.stack([s4, s4 + 1, s4 + 2, s4 + 3], axis=0)


def _idx_prep(src2d):
    return pl.pallas_call(
        _idx_prep_body,
        grid=(E_COLS // E_CBLK,),
        in_specs=[pl.BlockSpec((E_ROWS, E_CBLK), lambda i: (0, i))],
        out_specs=[
            pl.BlockSpec((2, E_ROWS, E_CBLK), lambda i: (0, 0, i)),
            pl.BlockSpec((4, E_ROWS, E_CBLK), lambda i: (0, 0, i)),
        ],
        out_shape=[
            jax.ShapeDtypeStruct((2, E_ROWS, E_COLS), jnp.int32),
            jax.ShapeDtypeStruct((4, E_ROWS, E_COLS), jnp.int32),
        ],
    )(src2d)


def _layer1_body(agg_ref, x2_ref, dis_ref, w_ref, b_ref, g_ref,
                 be_ref, o_ref):
    d = dis_ref[...]
    w = w_ref[...]
    h = jnp.dot(x2_ref[...] * d, w, preferred_element_type=jnp.float32)
    for c in range(2):
        h = h + jnp.dot(agg_ref[c] * d, w[16 * c:16 * c + 16, :],
                        preferred_element_type=jnp.float32)
    h = h + b_ref[...]
    o_ref[...] = _ln(h, g_ref[...], be_ref[...]) * d


def _layer1(agg1, x2, dis, W1, b1, g1, be1):
    return pl.pallas_call(
        _layer1_body,
        grid=(GRID_N,),
        in_specs=[
            pl.BlockSpec((2, BLK, 16), lambda i: (0, i, 0)),
            pl.BlockSpec((BLK, D_IN), lambda i: (i, 0)),
            pl.BlockSpec((BLK, 1), lambda i: (i, 0)),
            pl.BlockSpec((D_IN, H), lambda i: (0, 0)),
            pl.BlockSpec((1, H), lambda i: (0, 0)),
            pl.BlockSpec((1, H), lambda i: (0, 0)),
            pl.BlockSpec((1, H), lambda i: (0, 0)),
        ],
        out_specs=pl.BlockSpec((BLK, H), lambda i: (i, 0)),
        out_shape=_f32(N_PAD, H),
    )(agg1, x2, dis, W1, b1.reshape(1, -1), g1.reshape(1, -1),
      be1.reshape(1, -1))


def _layer2_body(agg_ref, hp_ref, dis_ref, w_ref, b_ref, g_ref,
                 be_ref, out_ref):
    d = dis_ref[...]
    w = w_ref[...]
    h = jnp.dot(hp_ref[...] * d, w, preferred_element_type=jnp.float32)
    for c in range(4):
        h = h + jnp.dot(agg_ref[c] * d, w[16 * c:16 * c + 16, :],
                        preferred_element_type=jnp.float32)
    h = h + b_ref[...]
    out_ref[...] = _ln(h, g_ref[...], be_ref[...])


def _layer2(agg2, hps, dis, W2, b2, g2, be2):
    return pl.pallas_call(
        _layer2_body,
        grid=(GRID_N,),
        in_specs=[
            pl.BlockSpec((4, BLK, 16), lambda i: (0, i, 0)),
            pl.BlockSpec((BLK, H), lambda i: (i, 0)),
            pl.BlockSpec((BLK, 1), lambda i: (i, 0)),
            pl.BlockSpec((H, H), lambda i: (0, 0)),
            pl.BlockSpec((1, H), lambda i: (0, 0)),
            pl.BlockSpec((1, H), lambda i: (0, 0)),
            pl.BlockSpec((1, H), lambda i: (0, 0)),
        ],
        out_specs=pl.BlockSpec((BLK, H), lambda i: (i, 0)),
        out_shape=_f32(N_PAD, H),
    )(agg2, hps, dis, W2, b2.reshape(1, -1), g2.reshape(1, -1),
      be2.reshape(1, -1))


MBLK = 512


def _head_body(sums_ref, cnts_ref, wr1_ref, br1_ref, wr2_ref, br2_ref,
               go_ref, bo_ref, out_ref):
    s = sums_ref[0] + sums_ref[1]
    cnt = cnts_ref[0, :] + cnts_ref[1, :]
    mean = s * jnp.reshape(1.0 / jnp.maximum(cnt, 1.0), (MBLK, 1))
    pooled = jnp.concatenate([mean, s], axis=-1)
    r = jnp.maximum(
        jnp.dot(pooled, wr1_ref[...], preferred_element_type=jnp.float32)
        + br1_ref[...], 0.0)
    o = jnp.dot(r, wr2_ref[...], preferred_element_type=jnp.float32) + br2_ref[...]
    out_ref[...] = _ln(o, go_ref[...], bo_ref[...])


def _head(sums, cnts, Wr1, br1, Wr2, br2, go, bo):
    return pl.pallas_call(
        _head_body,
        grid=(M // MBLK,),
        in_specs=[
            pl.BlockSpec((NC, MBLK, H), lambda i: (0, i, 0)),
            pl.BlockSpec((NC, MBLK), lambda i: (0, i)),
            pl.BlockSpec((2 * H, H), lambda i: (0, 0)),
            pl.BlockSpec((1, H), lambda i: (0, 0)),
            pl.BlockSpec((H, D_OUT), lambda i: (0, 0)),
            pl.BlockSpec((1, D_OUT), lambda i: (0, 0)),
            pl.BlockSpec((1, D_OUT), lambda i: (0, 0)),
            pl.BlockSpec((1, D_OUT), lambda i: (0, 0)),
        ],
        out_specs=pl.BlockSpec((MBLK, D_OUT), lambda i: (i, 0)),
        out_shape=_f32(M, D_OUT),
    )(sums, cnts, Wr1, br1.reshape(1, -1), Wr2, br2.reshape(1, -1),
      go.reshape(1, -1), bo.reshape(1, -1))


def kernel(x, edge_index, batch_ids, W1, b1, g1, be1, W2, b2, g2, be2,
           Wr1, br1, Wr2, br2, go, bo):
    src = edge_index[0]
    bid_pad = jnp.concatenate(
        [batch_ids, jnp.full((N_PAD - N,), M, jnp.int32)])

    zeros_n1 = jnp.zeros((N_PAD,), jnp.float32)
    zeros_n16 = jnp.zeros((N_PAD, 16), jnp.float32)
    zeros_m2 = jnp.zeros((M_PAD, H), jnp.float32)
    zeros_m1 = jnp.zeros((M_PAD,), jnp.float32)
    ones = jnp.ones((DEG_K,), jnp.float32)

    i2, i4 = _idx_prep(src.reshape(E_ROWS, E_COLS))
    i2 = i2.reshape(2, E)
    i4 = i4.reshape(4, E)
    counts = _deg_kernel(edge_index, zeros_n1, ones)
    x2, dis = _prep(counts, x)
    agg1 = _agg2(i2, edge_index, zeros_n16, x2.reshape(2 * N_PAD, 16))
    hp = _layer1(agg1, x2, dis, W1, b1, g1, be1)
    agg2 = _agg4(i4, edge_index, zeros_n16, hp.reshape(4 * N_PAD, 16))
    h2 = _layer2(agg2, hp, dis, W2, b2, g2, be2)
    sums, cnts = _pool_kernel(h2, bid_pad, zeros_m2, zeros_m1, ones)
    return _head(sums, cnts, Wr1, br1, Wr2, br2, go, bo)


# speedup vs baseline: 1.0189x; 1.0189x over previous
import functools

import jax
import jax.numpy as jnp
from jax import lax
from jax.experimental import pallas as pl
from jax.experimental.pallas import tpu as pltpu
from jax.experimental.pallas import tpu_sc as plsc

N = 100000
E = 1600000
M = 4096
D_IN = 32
H = 64
D_OUT = 128
EPS = 1e-5

NC = 2
NS = 16
NW = NC * NS

N_PAD = 100352
TILE_N = N_PAD // NS
M_PAD = 4224
TILE_M = M_PAD // NS

EDGE_K = 800
E_PER_TILE = E // NS
N_CHUNKS = E_PER_TILE // EDGE_K
DEG_K = 2000
E_PER_W = E // NW
POOL_ROWS = N_PAD // NW
POOL_K = 1568

_MESH = plsc.VectorSubcoreMesh(core_axis_name="c", subcore_axis_name="s")
_SC_PARAMS = pltpu.CompilerParams(use_tc_tiling_on_sc=False)


def _f32(*shape):
    return jax.ShapeDtypeStruct(shape, jnp.float32)


@functools.partial(
    pl.kernel,
    out_type=_f32(NC, N_PAD),
    mesh=_MESH,
    compiler_params=_SC_PARAMS,
    scratch_types=[
        pltpu.VMEM((DEG_K,), jnp.int32),
        pltpu.VMEM((DEG_K,), jnp.float32),
        pltpu.VMEM_SHARED((N_PAD,), jnp.float32),
    ],
)
def _deg_kernel(dst_hbm, zeros_hbm, ones_hbm, out_hbm, idx_v, ones_v, acc):
    c = lax.axis_index("c")
    s = lax.axis_index("s")
    w = c * NS + s
    pltpu.sync_copy(zeros_hbm.at[pl.ds(s * TILE_N, TILE_N)],
                    acc.at[pl.ds(s * TILE_N, TILE_N)])
    pltpu.sync_copy(ones_hbm.at[pl.ds(0, DEG_K)], ones_v)
    plsc.subcore_barrier()

    @pl.loop(0, E_PER_W // DEG_K)
    def _(i):
        base = w * E_PER_W + i * DEG_K
        pltpu.sync_copy(dst_hbm.at[pl.ds(base, DEG_K)], idx_v)
        pltpu.sync_copy(ones_v, acc.at[idx_v], add=True)

    plsc.subcore_barrier()
    pltpu.sync_copy(acc.at[pl.ds(s * TILE_N, TILE_N)],
                    out_hbm.at[c, pl.ds(s * TILE_N, TILE_N)])


def _make_edge_agg(n_passes):
    n_chunks = n_passes * NC

    @functools.partial(
        pl.kernel,
        out_type=_f32(n_chunks, N_PAD, 16),
        mesh=_MESH,
        compiler_params=_SC_PARAMS,
        scratch_types=[
            pltpu.VMEM((2, EDGE_K), jnp.int32),
            pltpu.VMEM((2, EDGE_K), jnp.int32),
            pltpu.VMEM((2, EDGE_K, 16), jnp.float32),
            pltpu.VMEM_SHARED((N_PAD, 16), jnp.float32),
            pltpu.SemaphoreType.DMA((2,)),
            pltpu.SemaphoreType.DMA((2,)),
            pltpu.SemaphoreType.DMA((2,)),
            pltpu.SemaphoreType.DMA((2,)),
        ],
    )
    def agg_kernel(idx_hbm, dst_hbm, zeros_hbm, table_hbm, out_hbm,
                   sidx, didx, rows, acc, sem_si, sem_di, sem_g, sem_s):
        c = lax.axis_index("c")
        s = lax.axis_index("s")

        @pl.loop(0, n_passes)
        def _(p):
            k = c * n_passes + p
            table = table_hbm

            def base_of(i):
                return s * E_PER_TILE + jnp.minimum(i, N_CHUNKS - 1) * EDGE_K

            def idx_load(i, b):
                pltpu.async_copy(idx_hbm.at[k, pl.ds(base_of(i), EDGE_K)],
                                 sidx.at[b], sem_si.at[b])
                pltpu.async_copy(dst_hbm.at[pl.ds(base_of(i), EDGE_K)],
                                 didx.at[b], sem_di.at[b])
                pltpu.make_async_copy(idx_hbm.at[k, pl.ds(base_of(i), EDGE_K)],
                                      sidx.at[b], sem_si.at[b]).wait()
                pltpu.make_async_copy(dst_hbm.at[pl.ds(base_of(i), EDGE_K)],
                                      didx.at[b], sem_di.at[b]).wait()

            def g_start(b):
                pltpu.async_copy(table.at[sidx.at[b]], rows.at[b],
                                 sem_g.at[b])

            def g_wait(b):
                pltpu.make_async_copy(table.at[sidx.at[b]], rows.at[b],
                                      sem_g.at[b]).wait()

            def s_start(b):
                pltpu.async_copy(rows.at[b], acc.at[didx.at[b]],
                                 sem_s.at[b], add=True)

            def s_wait(b):
                pltpu.make_async_copy(rows.at[b], acc.at[didx.at[b]],
                                      sem_s.at[b]).wait()

            pltpu.sync_copy(zeros_hbm.at[pl.ds(s * TILE_N, TILE_N), :],
                            acc.at[pl.ds(s * TILE_N, TILE_N), :])
            plsc.subcore_barrier()
            pltpu.sync_copy(zeros_hbm.at[pl.ds(0, EDGE_K), :], rows.at[1])
            pltpu.sync_copy(dst_hbm.at[pl.ds(s * E_PER_TILE, EDGE_K)],
                            didx.at[1])
            s_start(1)
            idx_load(0, 0)
            g_start(0)

            @pl.loop(0, N_CHUNKS // 2)
            def _(j):
                i0 = 2 * j
                g_wait(0)
                s_start(0)
                s_wait(1)
                idx_load(i0 + 1, 1)
                g_start(1)
                g_wait(1)
                s_start(1)
                s_wait(0)
                idx_load(i0 + 2, 0)
                g_start(0)

            g_wait(0)
            s_start(0)
            s_wait(1)
            s_wait(0)
            plsc.subcore_barrier()
            pltpu.sync_copy(acc.at[pl.ds(s * TILE_N, TILE_N), :],
                            out_hbm.at[k, pl.ds(s * TILE_N, TILE_N), :])

    return agg_kernel


_agg2 = _make_edge_agg(1)
_agg4 = _make_edge_agg(2)


@functools.partial(
    pl.kernel,
    out_type=(_f32(NC, M_PAD, H), _f32(NC, M_PAD)),
    mesh=_MESH,
    compiler_params=_SC_PARAMS,
    scratch_types=[
        pltpu.VMEM((POOL_K,), jnp.int32),
        pltpu.VMEM((POOL_K, H), jnp.float32),
        pltpu.VMEM((DEG_K,), jnp.float32),
        pltpu.VMEM_SHARED((M_PAD, H), jnp.float32),
        pltpu.VMEM_SHARED((M_PAD,), jnp.float32),
    ],
)
def _pool_kernel(h2_hbm, bid_hbm, zeros2_hbm, zeros1_hbm, ones_hbm,
                 outs_hbm, outc_hbm, idx_v, rows_v, ones_v, accs, accc):
    c = lax.axis_index("c")
    s = lax.axis_index("s")
    w = c * NS + s
    pltpu.sync_copy(zeros2_hbm.at[pl.ds(s * TILE_M, TILE_M), :],
                    accs.at[pl.ds(s * TILE_M, TILE_M), :])
    pltpu.sync_copy(zeros1_hbm.at[pl.ds(s * TILE_M, TILE_M)],
                    accc.at[pl.ds(s * TILE_M, TILE_M)])
    pltpu.sync_copy(ones_hbm.at[pl.ds(0, DEG_K)], ones_v)
    plsc.subcore_barrier()

    @pl.loop(0, POOL_ROWS // POOL_K)
    def _(j):
        base = w * POOL_ROWS + j * POOL_K
        pltpu.sync_copy(h2_hbm.at[pl.ds(base, POOL_K), :], rows_v)
        pltpu.sync_copy(bid_hbm.at[pl.ds(base, POOL_K)], idx_v)
        pltpu.sync_copy(rows_v, accs.at[idx_v], add=True)
        pltpu.sync_copy(ones_v.at[pl.ds(0, POOL_K)], accc.at[idx_v], add=True)

    plsc.subcore_barrier()
    pltpu.sync_copy(accs.at[pl.ds(s * TILE_M, TILE_M), :],
                    outs_hbm.at[c, pl.ds(s * TILE_M, TILE_M), :])
    pltpu.sync_copy(accc.at[pl.ds(s * TILE_M, TILE_M)],
                    outc_hbm.at[c, pl.ds(s * TILE_M, TILE_M)])


BLK = 2048
GRID_N = N_PAD // BLK


def _ln(h, g, b):
    mu = jnp.mean(h, axis=-1, keepdims=True)
    var = jnp.mean((h - mu) ** 2, axis=-1, keepdims=True)
    return (h - mu) * lax.rsqrt(var + EPS) * g + b


def _prep_body(counts_ref, x_ref, x2_ref, dis_ref):
    cnt = counts_ref[0, :] + counts_ref[1, :]
    dis = lax.rsqrt(cnt + 1.0)
    d2 = jnp.reshape(dis, (BLK, 1))
    x2_ref[...] = x_ref[...] * d2
    dis_ref[...] = d2


def _prep(counts, x):
    return pl.pallas_call(
        _prep_body,
        grid=(GRID_N,),
        in_specs=[
            pl.BlockSpec((NC, BLK), lambda i: (0, i)),
            pl.BlockSpec((BLK, D_IN), lambda i: (i, 0)),
        ],
        out_specs=[
            pl.BlockSpec((BLK, D_IN), lambda i: (i, 0)),
            pl.BlockSpec((BLK, 1), lambda i: (i, 0)),
        ],
        out_shape=[_f32(N_PAD, D_IN), _f32(N_PAD, 1)],
    )(counts, x)


E_ROWS = 3125
E_COLS = 512
E_CBLK = 128


def _idx_prep_body(s_ref, i2_ref, i4_ref):
    s = s_ref[...]
    s2 = s * 2
    s4 = s * 4
    i2_ref[...] = jnp.stack([s2, s2 + 1], axis=0)
    i4_ref[...] = jnp.stack([s4, s4 + 1, s4 + 2, s4 + 3], axis=0)


def _idx_prep(src2d):
    return pl.pallas_call(
        _idx_prep_body,
        grid=(E_COLS // E_CBLK,),
        in_specs=[pl.BlockSpec((E_ROWS, E_CBLK), lambda i: (0, i))],
        out_specs=[
            pl.BlockSpec((2, E_ROWS, E_CBLK), lambda i: (0, 0, i)),
            pl.BlockSpec((4, E_ROWS, E_CBLK), lambda i: (0, 0, i)),
        ],
        out_shape=[
            jax.ShapeDtypeStruct((2, E_ROWS, E_COLS), jnp.int32),
            jax.ShapeDtypeStruct((4, E_ROWS, E_COLS), jnp.int32),
        ],
    )(src2d)


def _layer1_body(agg_ref, x2_ref, dis_ref, w_ref, b_ref, g_ref,
                 be_ref, o_ref):
    d = dis_ref[...]
    w = w_ref[...]
    h = jnp.dot(x2_ref[...] * d, w, preferred_element_type=jnp.float32)
    for c in range(2):
        h = h + jnp.dot(agg_ref[c] * d, w[16 * c:16 * c + 16, :],
                        preferred_element_type=jnp.float32)
    h = h + b_ref[...]
    o_ref[...] = _ln(h, g_ref[...], be_ref[...]) * d


def _layer1(agg1, x2, dis, W1, b1, g1, be1):
    return pl.pallas_call(
        _layer1_body,
        grid=(GRID_N,),
        in_specs=[
            pl.BlockSpec((2, BLK, 16), lambda i: (0, i, 0)),
            pl.BlockSpec((BLK, D_IN), lambda i: (i, 0)),
            pl.BlockSpec((BLK, 1), lambda i: (i, 0)),
            pl.BlockSpec((D_IN, H), lambda i: (0, 0)),
            pl.BlockSpec((1, H), lambda i: (0, 0)),
            pl.BlockSpec((1, H), lambda i: (0, 0)),
            pl.BlockSpec((1, H), lambda i: (0, 0)),
        ],
        out_specs=pl.BlockSpec((BLK, H), lambda i: (i, 0)),
        out_shape=_f32(N_PAD, H),
    )(agg1, x2, dis, W1, b1.reshape(1, -1), g1.reshape(1, -1),
      be1.reshape(1, -1))


def _layer2_body(agg_ref, hp_ref, dis_ref, w_ref, b_ref, g_ref,
                 be_ref, out_ref):
    d = dis_ref[...]
    w = w_ref[...]
    h = jnp.dot(hp_ref[...] * d, w, preferred_element_type=jnp.float32)
    for c in range(4):
        h = h + jnp.dot(agg_ref[c] * d, w[16 * c:16 * c + 16, :],
                        preferred_element_type=jnp.float32)
    h = h + b_ref[...]
    out_ref[...] = _ln(h, g_ref[...], be_ref[...])


def _layer2(agg2, hps, dis, W2, b2, g2, be2):
    return pl.pallas_call(
        _layer2_body,
        grid=(GRID_N,),
        in_specs=[
            pl.BlockSpec((4, BLK, 16), lambda i: (0, i, 0)),
            pl.BlockSpec((BLK, H), lambda i: (i, 0)),
            pl.BlockSpec((BLK, 1), lambda i: (i, 0)),
            pl.BlockSpec((H, H), lambda i: (0, 0)),
            pl.BlockSpec((1, H), lambda i: (0, 0)),
            pl.BlockSpec((1, H), lambda i: (0, 0)),
            pl.BlockSpec((1, H), lambda i: (0, 0)),
        ],
        out_specs=pl.BlockSpec((BLK, H), lambda i: (i, 0)),
        out_shape=_f32(N_PAD, H),
    )(agg2, hps, dis, W2, b2.reshape(1, -1), g2.reshape(1, -1),
      be2.reshape(1, -1))


MBLK = 512


def _head_body(sums_ref, cnts_ref, wr1_ref, br1_ref, wr2_ref, br2_ref,
               go_ref, bo_ref, out_ref):
    s = sums_ref[0] + sums_ref[1]
    cnt = cnts_ref[0, :] + cnts_ref[1, :]
    mean = s * jnp.reshape(1.0 / jnp.maximum(cnt, 1.0), (MBLK, 1))
    pooled = jnp.concatenate([mean, s], axis=-1)
    r = jnp.maximum(
        jnp.dot(pooled, wr1_ref[...], preferred_element_type=jnp.float32)
        + br1_ref[...], 0.0)
    o = jnp.dot(r, wr2_ref[...], preferred_element_type=jnp.float32) + br2_ref[...]
    out_ref[...] = _ln(o, go_ref[...], bo_ref[...])


def _head(sums, cnts, Wr1, br1, Wr2, br2, go, bo):
    return pl.pallas_call(
        _head_body,
        grid=(M // MBLK,),
        in_specs=[
            pl.BlockSpec((NC, MBLK, H), lambda i: (0, i, 0)),
            pl.BlockSpec((NC, MBLK), lambda i: (0, i)),
            pl.BlockSpec((2 * H, H), lambda i: (0, 0)),
            pl.BlockSpec((1, H), lambda i: (0, 0)),
            pl.BlockSpec((H, D_OUT), lambda i: (0, 0)),
            pl.BlockSpec((1, D_OUT), lambda i: (0, 0)),
            pl.BlockSpec((1, D_OUT), lambda i: (0, 0)),
            pl.BlockSpec((1, D_OUT), lambda i: (0, 0)),
        ],
        out_specs=pl.BlockSpec((MBLK, D_OUT), lambda i: (i, 0)),
        out_shape=_f32(M, D_OUT),
    )(sums, cnts, Wr1, br1.reshape(1, -1), Wr2, br2.reshape(1, -1),
      go.reshape(1, -1), bo.reshape(1, -1))


def kernel(x, edge_index, batch_ids, W1, b1, g1, be1, W2, b2, g2, be2,
           Wr1, br1, Wr2, br2, go, bo):
    src = edge_index[0]
    dst = edge_index[1]
    bid_pad = jnp.concatenate(
        [batch_ids, jnp.full((N_PAD - N,), M, jnp.int32)])

    zeros_n1 = jnp.zeros((N_PAD,), jnp.float32)
    zeros_n16 = jnp.zeros((N_PAD, 16), jnp.float32)
    zeros_m2 = jnp.zeros((M_PAD, H), jnp.float32)
    zeros_m1 = jnp.zeros((M_PAD,), jnp.float32)
    ones = jnp.ones((DEG_K,), jnp.float32)

    i2, i4 = _idx_prep(src.reshape(E_ROWS, E_COLS))
    i2 = i2.reshape(2, E)
    i4 = i4.reshape(4, E)
    counts = _deg_kernel(dst, zeros_n1, ones)
    x2, dis = _prep(counts, x)
    agg1 = _agg2(i2, dst, zeros_n16, x2.reshape(2 * N_PAD, 16))
    hp = _layer1(agg1, x2, dis, W1, b1, g1, be1)
    agg2 = _agg4(i4, dst, zeros_n16, hp.reshape(4 * N_PAD, 16))
    h2 = _layer2(agg2, hp, dis, W2, b2, g2, be2)
    sums, cnts = _pool_kernel(h2, bid_pad, zeros_m2, zeros_m1, ones)
    return _head(sums, cnts, Wr1, br1, Wr2, br2, go, bo)


# speedup vs baseline: 1.0289x; 1.0099x over previous
import functools

import jax
import jax.numpy as jnp
from jax import lax
from jax.experimental import pallas as pl
from jax.experimental.pallas import tpu as pltpu
from jax.experimental.pallas import tpu_sc as plsc

N = 100000
E = 1600000
M = 4096
D_IN = 32
H = 64
D_OUT = 128
EPS = 1e-5

NC = 2
NS = 16
NW = NC * NS

N_PAD = 100352
TILE_N = N_PAD // NS
M_PAD = 4224
TILE_M = M_PAD // NS

EDGE_K = 800
E_PER_TILE = E // NS
N_CHUNKS = E_PER_TILE // EDGE_K
DEG_K = 2000
E_PER_W = E // NW
POOL_ROWS = N_PAD // NW
POOL_K = 1568

_MESH = plsc.VectorSubcoreMesh(core_axis_name="c", subcore_axis_name="s")
_SC_PARAMS = pltpu.CompilerParams(use_tc_tiling_on_sc=False)


def _f32(*shape):
    return jax.ShapeDtypeStruct(shape, jnp.float32)


@functools.partial(
    pl.kernel,
    out_type=_f32(NC, N_PAD),
    mesh=_MESH,
    compiler_params=_SC_PARAMS,
    scratch_types=[
        pltpu.VMEM((DEG_K,), jnp.int32),
        pltpu.VMEM((DEG_K,), jnp.float32),
        pltpu.VMEM_SHARED((N_PAD,), jnp.float32),
    ],
)
def _deg_kernel(dst_hbm, zeros_hbm, ones_hbm, out_hbm, idx_v, ones_v, acc):
    c = lax.axis_index("c")
    s = lax.axis_index("s")
    w = c * NS + s
    pltpu.sync_copy(zeros_hbm.at[pl.ds(s * TILE_N, TILE_N)],
                    acc.at[pl.ds(s * TILE_N, TILE_N)])
    pltpu.sync_copy(ones_hbm.at[pl.ds(0, DEG_K)], ones_v)
    plsc.subcore_barrier()

    @pl.loop(0, E_PER_W // DEG_K)
    def _(i):
        base = w * E_PER_W + i * DEG_K
        pltpu.sync_copy(dst_hbm.at[pl.ds(base, DEG_K)], idx_v)
        pltpu.sync_copy(ones_v, acc.at[idx_v], add=True)

    plsc.subcore_barrier()
    pltpu.sync_copy(acc.at[pl.ds(s * TILE_N, TILE_N)],
                    out_hbm.at[c, pl.ds(s * TILE_N, TILE_N)])


def _make_edge_agg(n_passes):
    n_chunks = n_passes * NC

    @functools.partial(
        pl.kernel,
        out_type=_f32(n_chunks, N_PAD, 16),
        mesh=_MESH,
        compiler_params=_SC_PARAMS,
        scratch_types=[
            pltpu.VMEM((2, EDGE_K), jnp.int32),
            pltpu.VMEM((2, EDGE_K), jnp.int32),
            pltpu.VMEM((2, EDGE_K, 16), jnp.float32),
            pltpu.VMEM_SHARED((N_PAD, 16), jnp.float32),
            pltpu.SemaphoreType.DMA((2,)),
            pltpu.SemaphoreType.DMA((2,)),
            pltpu.SemaphoreType.DMA((2,)),
            pltpu.SemaphoreType.DMA((2,)),
        ],
    )
    def agg_kernel(idx_hbm, dst_hbm, zeros_hbm, table_hbm, out_hbm,
                   sidx, didx, rows, acc, sem_si, sem_di, sem_g, sem_s):
        c = lax.axis_index("c")
        s = lax.axis_index("s")

        @pl.loop(0, n_passes)
        def _(p):
            k = c * n_passes + p
            table = table_hbm

            def base_of(i):
                return s * E_PER_TILE + jnp.minimum(i, N_CHUNKS - 1) * EDGE_K

            def sidx_start(i, b):
                pltpu.async_copy(idx_hbm.at[k, pl.ds(base_of(i), EDGE_K)],
                                 sidx.at[b], sem_si.at[b])

            def sidx_wait(i, b):
                pltpu.make_async_copy(idx_hbm.at[k, pl.ds(base_of(i), EDGE_K)],
                                      sidx.at[b], sem_si.at[b]).wait()

            def didx_load(i, b):
                pltpu.async_copy(dst_hbm.at[pl.ds(base_of(i), EDGE_K)],
                                 didx.at[b], sem_di.at[b])
                pltpu.make_async_copy(dst_hbm.at[pl.ds(base_of(i), EDGE_K)],
                                      didx.at[b], sem_di.at[b]).wait()

            def idx_load(i, b):
                sidx_start(i, b)
                didx_load(i, b)
                sidx_wait(i, b)

            def g_start(b):
                pltpu.async_copy(table.at[sidx.at[b]], rows.at[b],
                                 sem_g.at[b])

            def g_wait(b):
                pltpu.make_async_copy(table.at[sidx.at[b]], rows.at[b],
                                      sem_g.at[b]).wait()

            def s_start(b):
                pltpu.async_copy(rows.at[b], acc.at[didx.at[b]],
                                 sem_s.at[b], add=True)

            def s_wait(b):
                pltpu.make_async_copy(rows.at[b], acc.at[didx.at[b]],
                                      sem_s.at[b]).wait()

            pltpu.sync_copy(zeros_hbm.at[pl.ds(s * TILE_N, TILE_N), :],
                            acc.at[pl.ds(s * TILE_N, TILE_N), :])
            plsc.subcore_barrier()
            pltpu.sync_copy(zeros_hbm.at[pl.ds(0, EDGE_K), :], rows.at[1])
            pltpu.sync_copy(dst_hbm.at[pl.ds(s * E_PER_TILE, EDGE_K)],
                            didx.at[1])
            s_start(1)
            idx_load(0, 0)
            g_start(0)

            @pl.loop(0, N_CHUNKS // 2)
            def _(j):
                i0 = 2 * j
                sidx_start(i0 + 1, 1)
                g_wait(0)
                s_start(0)
                s_wait(1)
                didx_load(i0 + 1, 1)
                sidx_wait(i0 + 1, 1)
                g_start(1)
                sidx_start(i0 + 2, 0)
                g_wait(1)
                s_start(1)
                s_wait(0)
                didx_load(i0 + 2, 0)
                sidx_wait(i0 + 2, 0)
                g_start(0)

            g_wait(0)
            s_start(0)
            s_wait(1)
            s_wait(0)
            plsc.subcore_barrier()
            pltpu.sync_copy(acc.at[pl.ds(s * TILE_N, TILE_N), :],
                            out_hbm.at[k, pl.ds(s * TILE_N, TILE_N), :])

    return agg_kernel


_agg2 = _make_edge_agg(1)
_agg4 = _make_edge_agg(2)


@functools.partial(
    pl.kernel,
    out_type=(_f32(NC, M_PAD, H), _f32(NC, M_PAD)),
    mesh=_MESH,
    compiler_params=_SC_PARAMS,
    scratch_types=[
        pltpu.VMEM((POOL_K,), jnp.int32),
        pltpu.VMEM((POOL_K, H), jnp.float32),
        pltpu.VMEM((DEG_K,), jnp.float32),
        pltpu.VMEM_SHARED((M_PAD, H), jnp.float32),
        pltpu.VMEM_SHARED((M_PAD,), jnp.float32),
    ],
)
def _pool_kernel(h2_hbm, bid_hbm, zeros2_hbm, zeros1_hbm, ones_hbm,
                 outs_hbm, outc_hbm, idx_v, rows_v, ones_v, accs, accc):
    c = lax.axis_index("c")
    s = lax.axis_index("s")
    w = c * NS + s
    pltpu.sync_copy(zeros2_hbm.at[pl.ds(s * TILE_M, TILE_M), :],
                    accs.at[pl.ds(s * TILE_M, TILE_M), :])
    pltpu.sync_copy(zeros1_hbm.at[pl.ds(s * TILE_M, TILE_M)],
                    accc.at[pl.ds(s * TILE_M, TILE_M)])
    pltpu.sync_copy(ones_hbm.at[pl.ds(0, DEG_K)], ones_v)
    plsc.subcore_barrier()

    @pl.loop(0, POOL_ROWS // POOL_K)
    def _(j):
        base = w * POOL_ROWS + j * POOL_K
        pltpu.sync_copy(h2_hbm.at[pl.ds(base, POOL_K), :], rows_v)
        pltpu.sync_copy(bid_hbm.at[pl.ds(base, POOL_K)], idx_v)
        pltpu.sync_copy(rows_v, accs.at[idx_v], add=True)
        pltpu.sync_copy(ones_v.at[pl.ds(0, POOL_K)], accc.at[idx_v], add=True)

    plsc.subcore_barrier()
    pltpu.sync_copy(accs.at[pl.ds(s * TILE_M, TILE_M), :],
                    outs_hbm.at[c, pl.ds(s * TILE_M, TILE_M), :])
    pltpu.sync_copy(accc.at[pl.ds(s * TILE_M, TILE_M)],
                    outc_hbm.at[c, pl.ds(s * TILE_M, TILE_M)])


BLK = 2048
GRID_N = N_PAD // BLK


def _ln(h, g, b):
    mu = jnp.mean(h, axis=-1, keepdims=True)
    var = jnp.mean((h - mu) ** 2, axis=-1, keepdims=True)
    return (h - mu) * lax.rsqrt(var + EPS) * g + b


def _prep_body(counts_ref, x_ref, x2_ref, dis_ref):
    cnt = counts_ref[0, :] + counts_ref[1, :]
    dis = lax.rsqrt(cnt + 1.0)
    d2 = jnp.reshape(dis, (BLK, 1))
    x2_ref[...] = x_ref[...] * d2
    dis_ref[...] = d2


def _prep(counts, x):
    return pl.pallas_call(
        _prep_body,
        grid=(GRID_N,),
        in_specs=[
            pl.BlockSpec((NC, BLK), lambda i: (0, i)),
            pl.BlockSpec((BLK, D_IN), lambda i: (i, 0)),
        ],
        out_specs=[
            pl.BlockSpec((BLK, D_IN), lambda i: (i, 0)),
            pl.BlockSpec((BLK, 1), lambda i: (i, 0)),
        ],
        out_shape=[_f32(N_PAD, D_IN), _f32(N_PAD, 1)],
    )(counts, x)


E_ROWS = 3125
E_COLS = 512
E_CBLK = 128


def _idx_prep_body(s_ref, i2_ref, i4_ref):
    s = s_ref[...]
    s2 = s * 2
    s4 = s * 4
    i2_ref[...] = jnp.stack([s2, s2 + 1], axis=0)
    i4_ref[...] = jnp.stack([s4, s4 + 1, s4 + 2, s4 + 3], axis=0)


def _idx_prep(src2d):
    return pl.pallas_call(
        _idx_prep_body,
        grid=(E_COLS // E_CBLK,),
        in_specs=[pl.BlockSpec((E_ROWS, E_CBLK), lambda i: (0, i))],
        out_specs=[
            pl.BlockSpec((2, E_ROWS, E_CBLK), lambda i: (0, 0, i)),
            pl.BlockSpec((4, E_ROWS, E_CBLK), lambda i: (0, 0, i)),
        ],
        out_shape=[
            jax.ShapeDtypeStruct((2, E_ROWS, E_COLS), jnp.int32),
            jax.ShapeDtypeStruct((4, E_ROWS, E_COLS), jnp.int32),
        ],
    )(src2d)


def _layer1_body(agg_ref, x2_ref, dis_ref, w_ref, b_ref, g_ref,
                 be_ref, o_ref):
    d = dis_ref[...]
    w = w_ref[...]
    h = jnp.dot(x2_ref[...] * d, w, preferred_element_type=jnp.float32)
    for c in range(2):
        h = h + jnp.dot(agg_ref[c] * d, w[16 * c:16 * c + 16, :],
                        preferred_element_type=jnp.float32)
    h = h + b_ref[...]
    o_ref[...] = _ln(h, g_ref[...], be_ref[...]) * d


def _layer1(agg1, x2, dis, W1, b1, g1, be1):
    return pl.pallas_call(
        _layer1_body,
        grid=(GRID_N,),
        in_specs=[
            pl.BlockSpec((2, BLK, 16), lambda i: (0, i, 0)),
            pl.BlockSpec((BLK, D_IN), lambda i: (i, 0)),
            pl.BlockSpec((BLK, 1), lambda i: (i, 0)),
            pl.BlockSpec((D_IN, H), lambda i: (0, 0)),
            pl.BlockSpec((1, H), lambda i: (0, 0)),
            pl.BlockSpec((1, H), lambda i: (0, 0)),
            pl.BlockSpec((1, H), lambda i: (0, 0)),
        ],
        out_specs=pl.BlockSpec((BLK, H), lambda i: (i, 0)),
        out_shape=_f32(N_PAD, H),
    )(agg1, x2, dis, W1, b1.reshape(1, -1), g1.reshape(1, -1),
      be1.reshape(1, -1))


def _layer2_body(agg_ref, hp_ref, dis_ref, w_ref, b_ref, g_ref,
                 be_ref, out_ref):
    d = dis_ref[...]
    w = w_ref[...]
    h = jnp.dot(hp_ref[...] * d, w, preferred_element_type=jnp.float32)
    for c in range(4):
        h = h + jnp.dot(agg_ref[c] * d, w[16 * c:16 * c + 16, :],
                        preferred_element_type=jnp.float32)
    h = h + b_ref[...]
    out_ref[...] = _ln(h, g_ref[...], be_ref[...])


def _layer2(agg2, hps, dis, W2, b2, g2, be2):
    return pl.pallas_call(
        _layer2_body,
        grid=(GRID_N,),
        in_specs=[
            pl.BlockSpec((4, BLK, 16), lambda i: (0, i, 0)),
            pl.BlockSpec((BLK, H), lambda i: (i, 0)),
            pl.BlockSpec((BLK, 1), lambda i: (i, 0)),
            pl.BlockSpec((H, H), lambda i: (0, 0)),
            pl.BlockSpec((1, H), lambda i: (0, 0)),
            pl.BlockSpec((1, H), lambda i: (0, 0)),
            pl.BlockSpec((1, H), lambda i: (0, 0)),
        ],
        out_specs=pl.BlockSpec((BLK, H), lambda i: (i, 0)),
        out_shape=_f32(N_PAD, H),
    )(agg2, hps, dis, W2, b2.reshape(1, -1), g2.reshape(1, -1),
      be2.reshape(1, -1))


MBLK = 512


def _head_body(sums_ref, cnts_ref, wr1_ref, br1_ref, wr2_ref, br2_ref,
               go_ref, bo_ref, out_ref):
    s = sums_ref[0] + sums_ref[1]
    cnt = cnts_ref[0, :] + cnts_ref[1, :]
    mean = s * jnp.reshape(1.0 / jnp.maximum(cnt, 1.0), (MBLK, 1))
    pooled = jnp.concatenate([mean, s], axis=-1)
    r = jnp.maximum(
        jnp.dot(pooled, wr1_ref[...], preferred_element_type=jnp.float32)
        + br1_ref[...], 0.0)
    o = jnp.dot(r, wr2_ref[...], preferred_element_type=jnp.float32) + br2_ref[...]
    out_ref[...] = _ln(o, go_ref[...], bo_ref[...])


def _head(sums, cnts, Wr1, br1, Wr2, br2, go, bo):
    return pl.pallas_call(
        _head_body,
        grid=(M // MBLK,),
        in_specs=[
            pl.BlockSpec((NC, MBLK, H), lambda i: (0, i, 0)),
            pl.BlockSpec((NC, MBLK), lambda i: (0, i)),
            pl.BlockSpec((2 * H, H), lambda i: (0, 0)),
            pl.BlockSpec((1, H), lambda i: (0, 0)),
            pl.BlockSpec((H, D_OUT), lambda i: (0, 0)),
            pl.BlockSpec((1, D_OUT), lambda i: (0, 0)),
            pl.BlockSpec((1, D_OUT), lambda i: (0, 0)),
            pl.BlockSpec((1, D_OUT), lambda i: (0, 0)),
        ],
        out_specs=pl.BlockSpec((MBLK, D_OUT), lambda i: (i, 0)),
        out_shape=_f32(M, D_OUT),
    )(sums, cnts, Wr1, br1.reshape(1, -1), Wr2, br2.reshape(1, -1),
      go.reshape(1, -1), bo.reshape(1, -1))


def kernel(x, edge_index, batch_ids, W1, b1, g1, be1, W2, b2, g2, be2,
           Wr1, br1, Wr2, br2, go, bo):
    src = edge_index[0]
    dst = edge_index[1]
    bid_pad = jnp.concatenate(
        [batch_ids, jnp.full((N_PAD - N,), M, jnp.int32)])

    zeros_n1 = jnp.zeros((N_PAD,), jnp.float32)
    zeros_n16 = jnp.zeros((N_PAD, 16), jnp.float32)
    zeros_m2 = jnp.zeros((M_PAD, H), jnp.float32)
    zeros_m1 = jnp.zeros((M_PAD,), jnp.float32)
    ones = jnp.ones((DEG_K,), jnp.float32)

    i2, i4 = _idx_prep(src.reshape(E_ROWS, E_COLS))
    i2 = i2.reshape(2, E)
    i4 = i4.reshape(4, E)
    counts = _deg_kernel(dst, zeros_n1, ones)
    x2, dis = _prep(counts, x)
    agg1 = _agg2(i2, dst, zeros_n16, x2.reshape(2 * N_PAD, 16))
    hp = _layer1(agg1, x2, dis, W1, b1, g1, be1)
    agg2 = _agg4(i4, dst, zeros_n16, hp.reshape(4 * N_PAD, 16))
    h2 = _layer2(agg2, hp, dis, W2, b2, g2, be2)
    sums, cnts = _pool_kernel(h2, bid_pad, zeros_m2, zeros_m1, ones)
    return _head(sums, cnts, Wr1, br1, Wr2, br2, go, bo)
